# baseline (speedup 1.0000x reference)
import jax
import jax.numpy as jnp
from jax.experimental import pallas as pl
from jax.experimental.pallas import tpu as pltpu

N = 10000
D_IN = 256
D_OUT = 256
BM = 200
NBLK = N // BM
DEPTH = 4
LOOK = 3


def _fused_kernel(adj_hbm, x_ref, w_ref, b_ref, out_ref, ring, y_ref, sems):
    i = pl.program_id(0)

    def start_copy(blk):
        slot = jax.lax.rem(blk, DEPTH)
        pltpu.make_async_copy(
            adj_hbm.at[pl.ds(blk * BM, BM), :],
            ring.at[slot],
            sems.at[slot],
        ).start()

    @pl.when(i == 0)
    def _prologue():
        for d in range(LOOK):
            start_copy(d)
        xb = x_ref[...].astype(jnp.bfloat16)
        wb = w_ref[...].astype(jnp.bfloat16)
        y = jnp.dot(xb, wb.T, preferred_element_type=jnp.float32)
        y_ref[...] = y.astype(jnp.bfloat16)

    @pl.when(i + LOOK < NBLK)
    def _lookahead():
        start_copy(i + LOOK)

    slot = jax.lax.rem(i, DEPTH)
    pltpu.make_async_copy(
        adj_hbm.at[pl.ds(i * BM, BM), :],
        ring.at[slot],
        sems.at[slot],
    ).wait()
    a = ring[slot].astype(jnp.bfloat16)
    acc = jnp.dot(a, y_ref[...], preferred_element_type=jnp.float32)
    out_ref[...] = jnp.maximum(acc + b_ref[...], 0.0)


def kernel(adj, x, W, b):
    b2 = b.reshape(1, D_OUT)
    return pl.pallas_call(
        _fused_kernel,
        grid=(NBLK,),
        in_specs=[
            pl.BlockSpec(memory_space=pl.ANY),
            pl.BlockSpec((N, D_IN), lambda i: (0, 0)),
            pl.BlockSpec((D_OUT, D_IN), lambda i: (0, 0)),
            pl.BlockSpec((1, D_OUT), lambda i: (0, 0)),
        ],
        out_specs=pl.BlockSpec((BM, D_OUT), lambda i: (i, 0)),
        out_shape=jax.ShapeDtypeStruct((N, D_OUT), jnp.float32),
        scratch_shapes=[
            pltpu.VMEM((DEPTH, BM, N), jnp.float32),
            pltpu.VMEM((N, D_OUT), jnp.bfloat16),
            pltpu.SemaphoreType.DMA((DEPTH,)),
        ],
    )(adj, x, W, b2)


# speedup vs baseline: 1.0007x; 1.0007x over previous
import jax
import jax.numpy as jnp
from jax.experimental import pallas as pl
from jax.experimental.pallas import tpu as pltpu

N = 10000
D_IN = 256
D_OUT = 256
BM = 400
NBLK = N // BM
DEPTH = 2
SPLIT = 5
BSUB = BM // SPLIT


def _fused_kernel(adj_hbm, x_ref, w_ref, b_ref, out_ref, ring, y_ref, sems):
    i = pl.program_id(0)

    def start_copy(blk):
        slot = jax.lax.rem(blk, DEPTH)
        for s in range(SPLIT):
            pltpu.make_async_copy(
                adj_hbm.at[pl.ds(blk * BM + s * BSUB, BSUB), :],
                ring.at[slot, pl.ds(s * BSUB, BSUB), :],
                sems.at[slot, s],
            ).start()

    def wait_copy(blk):
        slot = jax.lax.rem(blk, DEPTH)
        for s in range(SPLIT):
            pltpu.make_async_copy(
                adj_hbm.at[pl.ds(blk * BM + s * BSUB, BSUB), :],
                ring.at[slot, pl.ds(s * BSUB, BSUB), :],
                sems.at[slot, s],
            ).wait()

    @pl.when(i == 0)
    def _prologue():
        start_copy(0)
        xb = x_ref[...].astype(jnp.bfloat16)
        wb = w_ref[...].astype(jnp.bfloat16)
        y = jnp.dot(xb, wb.T, preferred_element_type=jnp.float32)
        y_ref[...] = y.astype(jnp.bfloat16)

    @pl.when(i + 1 < NBLK)
    def _lookahead():
        start_copy(i + 1)

    wait_copy(i)
    slot = jax.lax.rem(i, DEPTH)
    a = ring[slot].astype(jnp.bfloat16)
    acc = jnp.dot(a, y_ref[...], preferred_element_type=jnp.float32)
    out_ref[...] = jnp.maximum(acc + b_ref[...], 0.0)


def kernel(adj, x, W, b):
    b2 = b.reshape(1, D_OUT)
    return pl.pallas_call(
        _fused_kernel,
        grid=(NBLK,),
        in_specs=[
            pl.BlockSpec(memory_space=pl.ANY),
            pl.BlockSpec((N, D_IN), lambda i: (0, 0)),
            pl.BlockSpec((D_OUT, D_IN), lambda i: (0, 0)),
            pl.BlockSpec((1, D_OUT), lambda i: (0, 0)),
        ],
        out_specs=pl.BlockSpec((BM, D_OUT), lambda i: (i, 0)),
        out_shape=jax.ShapeDtypeStruct((N, D_OUT), jnp.float32),
        scratch_shapes=[
            pltpu.VMEM((DEPTH, BM, N), jnp.float32),
            pltpu.VMEM((N, D_OUT), jnp.bfloat16),
            pltpu.SemaphoreType.DMA((DEPTH, SPLIT)),
        ],
    )(adj, x, W, b2)
